# baseline (speedup 1.0000x reference)
import functools

import jax
import jax.numpy as jnp
from jax import lax
from jax.experimental import pallas as pl
from jax.experimental.pallas import tpu as pltpu
from jax.experimental.pallas import tpu_sc as plsc


def _make_gather(B, D, NC, NS):
    NW = NC * NS
    b_per_w = B // NW
    mesh = plsc.VectorSubcoreMesh(core_axis_name="c", subcore_axis_name="s")

    @functools.partial(
        pl.kernel,
        mesh=mesh,
        compiler_params=pltpu.CompilerParams(use_tc_tiling_on_sc=False),
        out_type=jax.ShapeDtypeStruct((B, D), jnp.float32),
        scratch_types=[
            pltpu.VMEM((b_per_w,), jnp.int32),
            pltpu.VMEM((b_per_w, D), jnp.float32),
            pltpu.SemaphoreType.DMA,
        ],
    )
    def gather(uid_hbm, ut_hbm, uout_hbm, uidx_v, urows_v, usem):
        wid = lax.axis_index("s") * NC + lax.axis_index("c")
        base = wid * b_per_w
        pltpu.sync_copy(uid_hbm.at[pl.ds(base, b_per_w)], uidx_v)
        pltpu.async_copy(ut_hbm.at[uidx_v], urows_v, usem).wait()
        pltpu.sync_copy(urows_v, uout_hbm.at[pl.ds(base, b_per_w)])

    return gather


def kernel(user_ids, movie_ids, user_table, movie_table, W1, b1, W2, b2, W3, b3):
    B = user_ids.shape[0]
    D = user_table.shape[1]
    pos = jax.lax.iota(jnp.int32, B)
    us_ids, us_mids, us_pos = jax.lax.sort(
        (user_ids, movie_ids, pos), num_keys=1)
    info = plsc.get_sparse_core_info()
    gather = _make_gather(B, D, info.num_cores, info.num_subcores)
    u_emb = gather(us_ids, user_table)
    return jnp.sum(u_emb, axis=1) + us_mids.astype(jnp.float32) + us_pos.astype(jnp.float32)


# speedup vs baseline: 2.1948x; 2.1948x over previous
import functools

import jax
import jax.numpy as jnp
from jax import lax
from jax.experimental import pallas as pl
from jax.experimental.pallas import tpu as pltpu
from jax.experimental.pallas import tpu_sc as plsc

_CH = 512


def _make_stream_gather(NU, D, NC, NS, b_per_w):
    NW = NC * NS
    mesh = plsc.VectorSubcoreMesh(core_axis_name="c", subcore_axis_name="s")
    n_grp = b_per_w // 16

    @functools.partial(
        pl.kernel,
        mesh=mesh,
        compiler_params=pltpu.CompilerParams(needs_layout_passes=False),
        out_type=jax.ShapeDtypeStruct((NW, D, b_per_w), jnp.float32),
        scratch_types=[
            pltpu.VMEM((b_per_w,), jnp.int32),
            pltpu.VMEM((2, D, _CH), jnp.float32),
            pltpu.VMEM((D, b_per_w), jnp.float32),
            pltpu.SemaphoreType.DMA,
        ],
    )
    def stream_gather(ids2_hbm, utt_hbm, out_hbm, ids_v, chunk_v, out_v, sem):
        wid = lax.axis_index("s") * NC + lax.axis_index("c")
        pltpu.sync_copy(ids2_hbm.at[wid], ids_v)
        iota = lax.iota(jnp.int32, 16)
        head = ids_v[pl.ds(0, 16)]
        tail = ids_v[pl.ds(b_per_w - 16, 16)]
        lo = jnp.min(head) // _CH
        hi = jnp.max(tail) // _CH

        pltpu.async_copy(utt_hbm.at[:, pl.ds(lo * _CH, _CH)],
                         chunk_v.at[lax.rem(lo, 2)], sem)

        def chunk_body(c, carry):
            @pl.when(c + 1 <= hi)
            def _():
                pltpu.async_copy(utt_hbm.at[:, pl.ds((c + 1) * _CH, _CH)],
                                 chunk_v.at[lax.rem(c + 1, 2)], sem)

            cur = chunk_v.at[lax.rem(c, 2)]
            pltpu.make_async_copy(utt_hbm.at[:, pl.ds(c * _CH, _CH)],
                                  cur, sem).wait()
            c0 = c * _CH
            c1 = c0 + _CH

            def grp_body(g, carry2):
                idg = ids_v[pl.ds(g * 16, 16)]
                gmin = jnp.min(idg)
                gmax = jnp.max(idg)

                @pl.when(jnp.logical_and(gmax >= c0, gmin < c1))
                def _():
                    mask = jnp.logical_and(idg >= c0, idg < c1)
                    rel = jnp.where(mask, idg - c0, 0)
                    slotv = g * 16 + iota
                    for f in range(D):
                        fv = iota * 0 + f
                        v = plsc.load_gather(cur, [fv, rel], mask=mask)
                        plsc.store_scatter(out_v, [fv, slotv], v, mask=mask)

                return carry2

            lax.fori_loop(0, n_grp, grp_body, 0)
            return carry

        lax.fori_loop(lo, hi + 1, chunk_body, 0)
        pltpu.sync_copy(out_v, out_hbm.at[wid])

    return stream_gather


def _make_movie_gather(D, NC, NS, b_per_w):
    NW = NC * NS
    mesh = plsc.VectorSubcoreMesh(core_axis_name="c", subcore_axis_name="s")
    n_grp = b_per_w // 16

    @functools.partial(
        pl.kernel,
        mesh=mesh,
        compiler_params=pltpu.CompilerParams(use_tc_tiling_on_sc=False,
                                             needs_layout_passes=False),
        out_type=jax.ShapeDtypeStruct((NW, D, b_per_w), jnp.float32),
        scratch_types=[
            pltpu.VMEM((b_per_w,), jnp.int32),
            pltpu.VMEM((b_per_w, D), jnp.float32),
            pltpu.VMEM((D, b_per_w), jnp.float32),
            pltpu.SemaphoreType.DMA,
        ],
    )
    def movie_gather(mids2_hbm, mt_hbm, out_hbm, idx_v, rows_v, t_v, sem):
        wid = lax.axis_index("s") * NC + lax.axis_index("c")
        pltpu.sync_copy(mids2_hbm.at[wid], idx_v)
        pltpu.async_copy(mt_hbm.at[idx_v], rows_v, sem).wait()
        iota = lax.iota(jnp.int32, 16)

        def grp_body(g, carry):
            rows16 = g * 16 + iota
            for f in range(D):
                fv = iota * 0 + f
                v = plsc.load_gather(rows_v, [rows16, fv])
                plsc.store_scatter(t_v, [fv, rows16], v)
            return carry

        lax.fori_loop(0, n_grp, grp_body, 0)
        pltpu.sync_copy(t_v, out_hbm.at[wid])

    return movie_gather


def _make_scatter(B, NC, NS, b_per_w):
    NW = NC * NS
    mesh = plsc.VectorSubcoreMesh(core_axis_name="c", subcore_axis_name="s")

    @functools.partial(
        pl.kernel,
        mesh=mesh,
        compiler_params=pltpu.CompilerParams(use_tc_tiling_on_sc=False),
        out_type=jax.ShapeDtypeStruct((B,), jnp.float32),
        scratch_types=[
            pltpu.VMEM((b_per_w,), jnp.int32),
            pltpu.VMEM((b_per_w,), jnp.float32),
            pltpu.SemaphoreType.DMA,
        ],
    )
    def scatter(res2_hbm, pos2_hbm, out_hbm, pos_v, val_v, sem):
        wid = lax.axis_index("s") * NC + lax.axis_index("c")
        pltpu.sync_copy(res2_hbm.at[wid], val_v)
        pltpu.sync_copy(pos2_hbm.at[wid], pos_v)
        pltpu.async_copy(val_v, out_hbm.at[pos_v], sem).wait()

    return scatter


def _mlp_t(u3, m3, W1uT, W1mT, b1c, W2T, b2c, w3c, b3, NW, D, b_per_w):
    H1 = W1uT.shape[0]
    H2 = W2T.shape[0]

    def body(u_ref, m_ref, w1u_ref, w1m_ref, b1_ref, w2_ref, b2_ref,
             w3_ref, b3_ref, o_ref):
        u = u_ref[0]
        m = m_ref[0]
        h1 = (jnp.dot(w1u_ref[...], u, preferred_element_type=jnp.float32)
              + jnp.dot(w1m_ref[...], m, preferred_element_type=jnp.float32)
              + b1_ref[...])
        h1 = jnp.maximum(h1, 0.0)
        h2 = jnp.maximum(
            jnp.dot(w2_ref[...], h1, preferred_element_type=jnp.float32)
            + b2_ref[...], 0.0)
        o = jnp.sum(h2 * w3_ref[...], axis=0) + b3_ref[0]
        o_ref[...] = o.reshape(1, b_per_w // 128, 128)

    out = pl.pallas_call(
        body,
        grid=(NW,),
        in_specs=[
            pl.BlockSpec((1, D, b_per_w), lambda w: (w, 0, 0)),
            pl.BlockSpec((1, D, b_per_w), lambda w: (w, 0, 0)),
            pl.BlockSpec((H1, D), lambda w: (0, 0)),
            pl.BlockSpec((H1, D), lambda w: (0, 0)),
            pl.BlockSpec((H1, 1), lambda w: (0, 0)),
            pl.BlockSpec((H2, H1), lambda w: (0, 0)),
            pl.BlockSpec((H2, 1), lambda w: (0, 0)),
            pl.BlockSpec((H2, 1), lambda w: (0, 0)),
            pl.BlockSpec(memory_space=pltpu.SMEM),
        ],
        out_specs=pl.BlockSpec((1, b_per_w // 128, 128), lambda w: (w, 0, 0)),
        out_shape=jax.ShapeDtypeStruct((NW, b_per_w // 128, 128), jnp.float32),
    )(u3, m3, W1uT, W1mT, b1c, W2T, b2c, w3c, b3)
    return out.reshape(NW * b_per_w)


def kernel(user_ids, movie_ids, user_table, movie_table, W1, b1, W2, b2, W3, b3):
    B = user_ids.shape[0]
    NU, D = user_table.shape
    info = plsc.get_sparse_core_info()
    NC, NS = info.num_cores, info.num_subcores
    NW = NC * NS
    b_per_w = B // NW

    pos = lax.iota(jnp.int32, B)
    us_ids, us_mids, us_pos = lax.sort((user_ids, movie_ids, pos), num_keys=1)

    u3 = _make_stream_gather(NU, D, NC, NS, b_per_w)(
        us_ids.reshape(NW, b_per_w), user_table.T)
    m3 = _make_movie_gather(D, NC, NS, b_per_w)(
        us_mids.reshape(NW, b_per_w), movie_table)

    res = _mlp_t(u3, m3, W1[:D].T, W1[D:].T, b1.reshape(-1, 1), W2.T,
                 b2.reshape(-1, 1), W3.reshape(1, -1).T, b3, NW, D, b_per_w)

    return _make_scatter(B, NC, NS, b_per_w)(
        res.reshape(NW, b_per_w), us_pos.reshape(NW, b_per_w))


# speedup vs baseline: 2.4593x; 1.1206x over previous
import functools

import jax
import jax.numpy as jnp
from jax import lax
from jax.experimental import pallas as pl
from jax.experimental.pallas import tpu as pltpu
from jax.experimental.pallas import tpu_sc as plsc

_CH = 512


def _make_stream_gather(NU, D, NC, NS, b_per_w):
    NW = NC * NS
    mesh = plsc.VectorSubcoreMesh(core_axis_name="c", subcore_axis_name="s")
    n_grp = b_per_w // 16

    @functools.partial(
        pl.kernel,
        mesh=mesh,
        compiler_params=pltpu.CompilerParams(needs_layout_passes=False),
        out_type=jax.ShapeDtypeStruct((NW, D, b_per_w), jnp.float32),
        scratch_types=[
            pltpu.VMEM((b_per_w,), jnp.int32),
            pltpu.VMEM((2, D, _CH), jnp.float32),
            pltpu.VMEM((D, b_per_w), jnp.float32),
            pltpu.SemaphoreType.DMA,
        ],
    )
    def stream_gather(ids2_hbm, utt_hbm, out_hbm, ids_v, chunk_v, out_v, sem):
        wid = lax.axis_index("s") * NC + lax.axis_index("c")
        pltpu.sync_copy(ids2_hbm.at[wid], ids_v)
        iota = lax.iota(jnp.int32, 16)
        head = ids_v[pl.ds(0, 16)]
        tail = ids_v[pl.ds(b_per_w - 16, 16)]
        lo = jnp.min(head) // _CH
        hi = jnp.max(tail) // _CH

        pltpu.async_copy(utt_hbm.at[:, pl.ds(lo * _CH, _CH)],
                         chunk_v.at[lax.rem(lo, 2)], sem)

        def chunk_body(c, carry):
            @pl.when(c + 1 <= hi)
            def _():
                pltpu.async_copy(utt_hbm.at[:, pl.ds((c + 1) * _CH, _CH)],
                                 chunk_v.at[lax.rem(c + 1, 2)], sem)

            cur = chunk_v.at[lax.rem(c, 2)]
            pltpu.make_async_copy(utt_hbm.at[:, pl.ds(c * _CH, _CH)],
                                  cur, sem).wait()
            c0 = c * _CH
            c1 = c0 + _CH

            def grp_body(g, carry2):
                idg = ids_v[pl.ds(g * 16, 16)]
                gmin = jnp.min(idg)
                gmax = jnp.max(idg)

                @pl.when(jnp.logical_and(gmax >= c0, gmin < c1))
                def _():
                    mask = jnp.logical_and(idg >= c0, idg < c1)
                    rel = jnp.where(mask, idg - c0, 0)
                    slotv = g * 16 + iota
                    for f in range(D):
                        fv = iota * 0 + f
                        v = plsc.load_gather(cur, [fv, rel], mask=mask)
                        plsc.store_scatter(out_v, [fv, slotv], v, mask=mask)

                return carry2

            lax.fori_loop(0, n_grp, grp_body, 0)
            return carry

        lax.fori_loop(lo, hi + 1, chunk_body, 0)
        pltpu.sync_copy(out_v, out_hbm.at[wid])

    return stream_gather


def _make_movie_gather(D, NC, NS, b_per_w):
    NW = NC * NS
    mesh = plsc.VectorSubcoreMesh(core_axis_name="c", subcore_axis_name="s")
    n_grp = b_per_w // 16

    @functools.partial(
        pl.kernel,
        mesh=mesh,
        compiler_params=pltpu.CompilerParams(use_tc_tiling_on_sc=False,
                                             needs_layout_passes=False),
        out_type=jax.ShapeDtypeStruct((NW, D, b_per_w), jnp.float32),
        scratch_types=[
            pltpu.VMEM((b_per_w,), jnp.int32),
            pltpu.VMEM((b_per_w, D), jnp.float32),
            pltpu.VMEM((D, b_per_w), jnp.float32),
            pltpu.SemaphoreType.DMA,
        ],
    )
    def movie_gather(mids2_hbm, mt_hbm, out_hbm, idx_v, rows_v, t_v, sem):
        wid = lax.axis_index("s") * NC + lax.axis_index("c")
        pltpu.sync_copy(mids2_hbm.at[wid], idx_v)
        pltpu.async_copy(mt_hbm.at[idx_v], rows_v, sem).wait()
        iota = lax.iota(jnp.int32, 16)

        def grp_body(g, carry):
            rows16 = g * 16 + iota
            for f in range(D):
                fv = iota * 0 + f
                v = plsc.load_gather(rows_v, [rows16, fv])
                plsc.store_scatter(t_v, [fv, rows16], v)
            return carry

        lax.fori_loop(0, n_grp, grp_body, 0)
        pltpu.sync_copy(t_v, out_hbm.at[wid])

    return movie_gather


def _mlp_t(u3, m3, W1uT, W1mT, b1c, W2T, b2c, w3c, b3, NW, D, b_per_w):
    H1 = W1uT.shape[0]
    H2 = W2T.shape[0]

    def body(u_ref, m_ref, w1u_ref, w1m_ref, b1_ref, w2_ref, b2_ref,
             w3_ref, b3_ref, o_ref):
        u = u_ref[0]
        m = m_ref[0]
        h1 = (jnp.dot(w1u_ref[...], u, preferred_element_type=jnp.float32)
              + jnp.dot(w1m_ref[...], m, preferred_element_type=jnp.float32)
              + b1_ref[...])
        h1 = jnp.maximum(h1, 0.0)
        h2 = jnp.maximum(
            jnp.dot(w2_ref[...], h1, preferred_element_type=jnp.float32)
            + b2_ref[...], 0.0)
        o = jnp.sum(h2 * w3_ref[...], axis=0) + b3_ref[0]
        o_ref[...] = o.reshape(1, b_per_w // 128, 128)

    out = pl.pallas_call(
        body,
        grid=(NW,),
        in_specs=[
            pl.BlockSpec((1, D, b_per_w), lambda w: (w, 0, 0)),
            pl.BlockSpec((1, D, b_per_w), lambda w: (w, 0, 0)),
            pl.BlockSpec((H1, D), lambda w: (0, 0)),
            pl.BlockSpec((H1, D), lambda w: (0, 0)),
            pl.BlockSpec((H1, 1), lambda w: (0, 0)),
            pl.BlockSpec((H2, H1), lambda w: (0, 0)),
            pl.BlockSpec((H2, 1), lambda w: (0, 0)),
            pl.BlockSpec((H2, 1), lambda w: (0, 0)),
            pl.BlockSpec(memory_space=pltpu.SMEM),
        ],
        out_specs=pl.BlockSpec((1, b_per_w // 128, 128), lambda w: (w, 0, 0)),
        out_shape=jax.ShapeDtypeStruct((NW, b_per_w // 128, 128), jnp.float32),
    )(u3, m3, W1uT, W1mT, b1c, W2T, b2c, w3c, b3)
    return out.reshape(NW * b_per_w)


def kernel(user_ids, movie_ids, user_table, movie_table, W1, b1, W2, b2, W3, b3):
    B = user_ids.shape[0]
    NU, D = user_table.shape
    info = plsc.get_sparse_core_info()
    NC, NS = info.num_cores, info.num_subcores
    NW = NC * NS
    b_per_w = B // NW

    pos = lax.iota(jnp.int32, B)
    us_ids, us_mids, us_pos = lax.sort((user_ids, movie_ids, pos), num_keys=1)

    u3 = _make_stream_gather(NU, D, NC, NS, b_per_w)(
        us_ids.reshape(NW, b_per_w), user_table.T)
    m3 = _make_movie_gather(D, NC, NS, b_per_w)(
        us_mids.reshape(NW, b_per_w), movie_table)

    res = _mlp_t(u3, m3, W1[:D].T, W1[D:].T, b1.reshape(-1, 1), W2.T,
                 b2.reshape(-1, 1), W3.reshape(1, -1).T, b3, NW, D, b_per_w)

    _, out = lax.sort((us_pos, res), num_keys=1)
    return out


# speedup vs baseline: 2.5547x; 1.0388x over previous
import functools

import jax
import jax.numpy as jnp
from jax import lax
from jax.experimental import pallas as pl
from jax.experimental.pallas import tpu as pltpu
from jax.experimental.pallas import tpu_sc as plsc

_CH = 640


def _make_stream_gather(NU, D, NC, NS, b_per_w):
    NW = NC * NS
    mesh = plsc.VectorSubcoreMesh(core_axis_name="c", subcore_axis_name="s")
    n_grp = b_per_w // 16

    @functools.partial(
        pl.kernel,
        mesh=mesh,
        compiler_params=pltpu.CompilerParams(needs_layout_passes=False),
        out_type=jax.ShapeDtypeStruct((NW, D, b_per_w), jnp.float32),
        scratch_types=[
            pltpu.VMEM((b_per_w,), jnp.int32),
            pltpu.VMEM((2, D, _CH), jnp.float32),
            pltpu.VMEM((D, b_per_w), jnp.float32),
            pltpu.SemaphoreType.DMA,
        ],
    )
    def stream_gather(ids2_hbm, utt_hbm, out_hbm, ids_v, chunk_v, out_v, sem):
        wid = lax.axis_index("s") * NC + lax.axis_index("c")
        pltpu.sync_copy(ids2_hbm.at[wid], ids_v)
        iota = lax.iota(jnp.int32, 16)
        head = ids_v[pl.ds(0, 16)]
        tail = ids_v[pl.ds(b_per_w - 16, 16)]
        lo = jnp.min(head) // _CH
        hi = jnp.max(tail) // _CH

        pltpu.async_copy(utt_hbm.at[:, pl.ds(lo * _CH, _CH)],
                         chunk_v.at[lax.rem(lo, 2)], sem)

        def chunk_body(c, carry):
            @pl.when(c + 1 <= hi)
            def _():
                pltpu.async_copy(utt_hbm.at[:, pl.ds((c + 1) * _CH, _CH)],
                                 chunk_v.at[lax.rem(c + 1, 2)], sem)

            cur = chunk_v.at[lax.rem(c, 2)]
            pltpu.make_async_copy(utt_hbm.at[:, pl.ds(c * _CH, _CH)],
                                  cur, sem).wait()
            c0 = c * _CH
            c1 = c0 + _CH

            def grp_body(g, carry2):
                idg = ids_v[pl.ds(g * 16, 16)]
                gmin = jnp.min(idg)
                gmax = jnp.max(idg)

                @pl.when(jnp.logical_and(gmax >= c0, gmin < c1))
                def _():
                    mask = jnp.logical_and(idg >= c0, idg < c1)
                    rel = jnp.where(mask, idg - c0, 0)
                    slotv = g * 16 + iota
                    for f in range(D):
                        fv = iota * 0 + f
                        v = plsc.load_gather(cur, [fv, rel], mask=mask)
                        plsc.store_scatter(out_v, [fv, slotv], v, mask=mask)

                return carry2

            lax.fori_loop(0, n_grp, grp_body, 0)
            return carry

        lax.fori_loop(lo, hi + 1, chunk_body, 0)
        pltpu.sync_copy(out_v, out_hbm.at[wid])

    return stream_gather


def _make_movie_gather(D, NC, NS, b_per_w):
    NW = NC * NS
    mesh = plsc.VectorSubcoreMesh(core_axis_name="c", subcore_axis_name="s")
    n_grp = b_per_w // 16

    @functools.partial(
        pl.kernel,
        mesh=mesh,
        compiler_params=pltpu.CompilerParams(use_tc_tiling_on_sc=False,
                                             needs_layout_passes=False),
        out_type=jax.ShapeDtypeStruct((NW, D, b_per_w), jnp.float32),
        scratch_types=[
            pltpu.VMEM((b_per_w,), jnp.int32),
            pltpu.VMEM((b_per_w, D), jnp.float32),
            pltpu.VMEM((D, b_per_w), jnp.float32),
            pltpu.SemaphoreType.DMA,
        ],
    )
    def movie_gather(mids2_hbm, mt_hbm, out_hbm, idx_v, rows_v, t_v, sem):
        wid = lax.axis_index("s") * NC + lax.axis_index("c")
        pltpu.sync_copy(mids2_hbm.at[wid], idx_v)
        pltpu.async_copy(mt_hbm.at[idx_v], rows_v, sem).wait()
        iota = lax.iota(jnp.int32, 16)

        def grp_body(g, carry):
            rows16 = g * 16 + iota
            for f in range(D):
                fv = iota * 0 + f
                v = plsc.load_gather(rows_v, [rows16, fv])
                plsc.store_scatter(t_v, [fv, rows16], v)
            return carry

        lax.fori_loop(0, n_grp, grp_body, 0)
        pltpu.sync_copy(t_v, out_hbm.at[wid])

    return movie_gather


def _mlp_t(u3, m3, W1uT, W1mT, b1c, W2T, b2c, w3c, b3, NW, D, b_per_w):
    H1 = W1uT.shape[0]
    H2 = W2T.shape[0]

    def body(u_ref, m_ref, w1u_ref, w1m_ref, b1_ref, w2_ref, b2_ref,
             w3_ref, b3_ref, o_ref):
        u = u_ref[0]
        m = m_ref[0]
        h1 = (jnp.dot(w1u_ref[...], u, preferred_element_type=jnp.float32)
              + jnp.dot(w1m_ref[...], m, preferred_element_type=jnp.float32)
              + b1_ref[...])
        h1 = jnp.maximum(h1, 0.0)
        h2 = jnp.maximum(
            jnp.dot(w2_ref[...], h1, preferred_element_type=jnp.float32)
            + b2_ref[...], 0.0)
        o = jnp.sum(h2 * w3_ref[...], axis=0) + b3_ref[0]
        o_ref[...] = o.reshape(1, b_per_w // 128, 128)

    out = pl.pallas_call(
        body,
        grid=(NW,),
        in_specs=[
            pl.BlockSpec((1, D, b_per_w), lambda w: (w, 0, 0)),
            pl.BlockSpec((1, D, b_per_w), lambda w: (w, 0, 0)),
            pl.BlockSpec((H1, D), lambda w: (0, 0)),
            pl.BlockSpec((H1, D), lambda w: (0, 0)),
            pl.BlockSpec((H1, 1), lambda w: (0, 0)),
            pl.BlockSpec((H2, H1), lambda w: (0, 0)),
            pl.BlockSpec((H2, 1), lambda w: (0, 0)),
            pl.BlockSpec((H2, 1), lambda w: (0, 0)),
            pl.BlockSpec(memory_space=pltpu.SMEM),
        ],
        out_specs=pl.BlockSpec((1, b_per_w // 128, 128), lambda w: (w, 0, 0)),
        out_shape=jax.ShapeDtypeStruct((NW, b_per_w // 128, 128), jnp.float32),
    )(u3, m3, W1uT, W1mT, b1c, W2T, b2c, w3c, b3)
    return out.reshape(NW * b_per_w)


def kernel(user_ids, movie_ids, user_table, movie_table, W1, b1, W2, b2, W3, b3):
    B = user_ids.shape[0]
    NU, D = user_table.shape
    info = plsc.get_sparse_core_info()
    NC, NS = info.num_cores, info.num_subcores
    NW = NC * NS
    b_per_w = B // NW

    pos = lax.iota(jnp.int32, B)
    us_ids, us_mids, us_pos = lax.sort((user_ids, movie_ids, pos), num_keys=1)

    u3 = _make_stream_gather(NU, D, NC, NS, b_per_w)(
        us_ids.reshape(NW, b_per_w), user_table.T)
    m3 = _make_movie_gather(D, NC, NS, b_per_w)(
        us_mids.reshape(NW, b_per_w), movie_table)

    res = _mlp_t(u3, m3, W1[:D].T, W1[D:].T, b1.reshape(-1, 1), W2.T,
                 b2.reshape(-1, 1), W3.reshape(1, -1).T, b3, NW, D, b_per_w)

    _, out = lax.sort((us_pos, res), num_keys=1)
    return out


# speedup vs baseline: 2.5608x; 1.0024x over previous
import functools

import jax
import jax.numpy as jnp
from jax import lax
from jax.experimental import pallas as pl
from jax.experimental.pallas import tpu as pltpu
from jax.experimental.pallas import tpu_sc as plsc

_CH = 640


def _make_stream_gather(NU, D, NC, NS, b_per_w):
    NW = NC * NS
    mesh = plsc.VectorSubcoreMesh(core_axis_name="c", subcore_axis_name="s")
    n_grp = b_per_w // 16
    NU_pad = ((NU + 127) // 128) * 128

    def _base(c):
        return jnp.minimum(c * _CH, NU_pad - _CH)

    @functools.partial(
        pl.kernel,
        mesh=mesh,
        compiler_params=pltpu.CompilerParams(needs_layout_passes=False),
        out_type=jax.ShapeDtypeStruct((NW, D, b_per_w), jnp.float32),
        scratch_types=[
            pltpu.VMEM((b_per_w,), jnp.int32),
            pltpu.VMEM((2, D, _CH), jnp.float32),
            pltpu.VMEM((D, b_per_w), jnp.float32),
            pltpu.SemaphoreType.DMA,
        ],
    )
    def stream_gather(ids2_hbm, utt_hbm, out_hbm, ids_v, chunk_v, out_v, sem):
        wid = lax.axis_index("s") * NC + lax.axis_index("c")
        pltpu.sync_copy(ids2_hbm.at[wid], ids_v)
        iota = lax.iota(jnp.int32, 16)
        head = ids_v[pl.ds(0, 16)]
        tail = ids_v[pl.ds(b_per_w - 16, 16)]
        lo = jnp.min(head) // _CH
        hi = jnp.max(tail) // _CH

        pltpu.async_copy(utt_hbm.at[:, pl.ds(_base(lo), _CH)],
                         chunk_v.at[lax.rem(lo, 2)], sem)

        def chunk_body(c, carry):
            @pl.when(c + 1 <= hi)
            def _():
                pltpu.async_copy(utt_hbm.at[:, pl.ds(_base(c + 1), _CH)],
                                 chunk_v.at[lax.rem(c + 1, 2)], sem)

            cur = chunk_v.at[lax.rem(c, 2)]
            base = _base(c)
            pltpu.make_async_copy(utt_hbm.at[:, pl.ds(base, _CH)],
                                  cur, sem).wait()
            c0 = c * _CH
            c1 = c0 + _CH

            def grp_body(g, carry2):
                idg = ids_v[pl.ds(g * 16, 16)]
                gmin = jnp.min(idg)
                gmax = jnp.max(idg)

                @pl.when(jnp.logical_and(gmax >= c0, gmin < c1))
                def _():
                    mask = jnp.logical_and(idg >= c0, idg < c1)
                    rel = jnp.where(mask, idg - base, 0)
                    slotv = g * 16 + iota
                    for f in range(D):
                        fv = iota * 0 + f
                        v = plsc.load_gather(cur, [fv, rel], mask=mask)
                        plsc.store_scatter(out_v, [fv, slotv], v, mask=mask)

                return carry2

            lax.fori_loop(0, n_grp, grp_body, 0)
            return carry

        lax.fori_loop(lo, hi + 1, chunk_body, 0)
        pltpu.sync_copy(out_v, out_hbm.at[wid])

    return stream_gather


def _make_movie_gather(D, NC, NS, b_per_w):
    NW = NC * NS
    mesh = plsc.VectorSubcoreMesh(core_axis_name="c", subcore_axis_name="s")
    n_grp = b_per_w // 16

    @functools.partial(
        pl.kernel,
        mesh=mesh,
        compiler_params=pltpu.CompilerParams(use_tc_tiling_on_sc=False,
                                             needs_layout_passes=False),
        out_type=jax.ShapeDtypeStruct((NW, D, b_per_w), jnp.float32),
        scratch_types=[
            pltpu.VMEM((b_per_w,), jnp.int32),
            pltpu.VMEM((b_per_w, D), jnp.float32),
            pltpu.VMEM((D, b_per_w), jnp.float32),
            pltpu.SemaphoreType.DMA,
        ],
    )
    def movie_gather(mids2_hbm, mt_hbm, out_hbm, idx_v, rows_v, t_v, sem):
        wid = lax.axis_index("s") * NC + lax.axis_index("c")
        pltpu.sync_copy(mids2_hbm.at[wid], idx_v)
        pltpu.async_copy(mt_hbm.at[idx_v], rows_v, sem).wait()
        iota = lax.iota(jnp.int32, 16)

        def grp_body(g, carry):
            rows16 = g * 16 + iota
            for f in range(D):
                fv = iota * 0 + f
                v = plsc.load_gather(rows_v, [rows16, fv])
                plsc.store_scatter(t_v, [fv, rows16], v)
            return carry

        lax.fori_loop(0, n_grp, grp_body, 0)
        pltpu.sync_copy(t_v, out_hbm.at[wid])

    return movie_gather


def _mlp_t(u3, m3, W1uT, W1mT, b1c, W2T, b2c, w3c, b3, NW, D, b_per_w):
    H1 = W1uT.shape[0]
    H2 = W2T.shape[0]

    def body(u_ref, m_ref, w1u_ref, w1m_ref, b1_ref, w2_ref, b2_ref,
             w3_ref, b3_ref, o_ref):
        u = u_ref[0]
        m = m_ref[0]
        h1 = (jnp.dot(w1u_ref[...], u, preferred_element_type=jnp.float32)
              + jnp.dot(w1m_ref[...], m, preferred_element_type=jnp.float32)
              + b1_ref[...])
        h1 = jnp.maximum(h1, 0.0)
        h2 = jnp.maximum(
            jnp.dot(w2_ref[...], h1, preferred_element_type=jnp.float32)
            + b2_ref[...], 0.0)
        o = jnp.sum(h2 * w3_ref[...], axis=0) + b3_ref[0]
        o_ref[...] = o.reshape(1, b_per_w // 128, 128)

    out = pl.pallas_call(
        body,
        grid=(NW,),
        in_specs=[
            pl.BlockSpec((1, D, b_per_w), lambda w: (w, 0, 0)),
            pl.BlockSpec((1, D, b_per_w), lambda w: (w, 0, 0)),
            pl.BlockSpec((H1, D), lambda w: (0, 0)),
            pl.BlockSpec((H1, D), lambda w: (0, 0)),
            pl.BlockSpec((H1, 1), lambda w: (0, 0)),
            pl.BlockSpec((H2, H1), lambda w: (0, 0)),
            pl.BlockSpec((H2, 1), lambda w: (0, 0)),
            pl.BlockSpec((H2, 1), lambda w: (0, 0)),
            pl.BlockSpec(memory_space=pltpu.SMEM),
        ],
        out_specs=pl.BlockSpec((1, b_per_w // 128, 128), lambda w: (w, 0, 0)),
        out_shape=jax.ShapeDtypeStruct((NW, b_per_w // 128, 128), jnp.float32),
    )(u3, m3, W1uT, W1mT, b1c, W2T, b2c, w3c, b3)
    return out.reshape(NW * b_per_w)


def kernel(user_ids, movie_ids, user_table, movie_table, W1, b1, W2, b2, W3, b3):
    B = user_ids.shape[0]
    NU, D = user_table.shape
    info = plsc.get_sparse_core_info()
    NC, NS = info.num_cores, info.num_subcores
    NW = NC * NS
    b_per_w = B // NW

    pos = lax.iota(jnp.int32, B)
    us_ids, us_mids, us_pos = lax.sort((user_ids, movie_ids, pos), num_keys=1)

    u3 = _make_stream_gather(NU, D, NC, NS, b_per_w)(
        us_ids.reshape(NW, b_per_w), user_table.T)
    m3 = _make_movie_gather(D, NC, NS, b_per_w)(
        us_mids.reshape(NW, b_per_w), movie_table)

    res = _mlp_t(u3, m3, W1[:D].T, W1[D:].T, b1.reshape(-1, 1), W2.T,
                 b2.reshape(-1, 1), W3.reshape(1, -1).T, b3, NW, D, b_per_w)

    _, out = lax.sort((us_pos, res), num_keys=1)
    return out
